# baseline (speedup 1.0000x reference)
import functools

import jax
import jax.numpy as jnp
import numpy as np
from jax import lax
from jax.experimental import pallas as pl
from jax.experimental.pallas import tpu as pltpu
from jax.experimental.pallas import tpu_sc as plsc

N_NODES = 10000
N_PAD = 10240
N_TILES = 16
ROWS_PER_TILE = N_PAD // N_TILES
N_EDGES = 160000
E_PER_TILE = N_EDGES // N_TILES


def _make_seg_sum(d_feat, chunk, with_counts):
  steps = E_PER_TILE // chunk
  assert steps * chunk == E_PER_TILE and chunk % 8 == 0
  mesh = plsc.VectorSubcoreMesh(core_axis_name="c", subcore_axis_name="s",
                                num_cores=2, num_subcores=N_TILES)
  out_type = [jax.ShapeDtypeStruct((N_PAD, d_feat), jnp.float32),
              jax.ShapeDtypeStruct((N_PAD, d_feat), jnp.float32)]
  if with_counts:
    out_type += [jax.ShapeDtypeStruct((N_PAD,), jnp.float32),
                 jax.ShapeDtypeStruct((N_PAD,), jnp.float32)]

  scratch = [
      pltpu.VMEM((chunk,), jnp.int32),
      pltpu.VMEM((chunk,), jnp.int32),
      pltpu.VMEM((chunk,), jnp.int32),
      pltpu.VMEM((chunk,), jnp.int32),
      pltpu.VMEM((chunk, d_feat), jnp.float32),
      pltpu.VMEM((chunk, d_feat), jnp.float32),
  ]
  if with_counts:
    scratch.append(pltpu.VMEM((chunk,), jnp.float32))
  scratch.append(pltpu.VMEM_SHARED((N_PAD, d_feat), jnp.float32))
  if with_counts:
    scratch.append(pltpu.VMEM_SHARED((N_PAD,), jnp.float32))
  n_sem = 10 if with_counts else 8
  scratch += [pltpu.SemaphoreType.DMA] * n_sem

  def body(*args):
    if with_counts:
      (featp, featn, z2, z1, ones_hbm, pe, ne,
       sum_p, sum_n, cnt_p, cnt_n,
       is0, is1, id0, id1, rw0, rw1, ones, acc, cacc,
       sem_is0, sem_is1, sem_id0, sem_id1,
       sem_g0, sem_g1, sem_s0, sem_s1, sem_c0, sem_c1) = args
      sem_c = (sem_c0, sem_c1)
    else:
      (featp, featn, z2, pe, ne,
       sum_p, sum_n,
       is0, is1, id0, id1, rw0, rw1, acc,
       sem_is0, sem_is1, sem_id0, sem_id1,
       sem_g0, sem_g1, sem_s0, sem_s1) = args
    idx_s = (is0, is1)
    idx_d = (id0, id1)
    rows = (rw0, rw1)
    sem_is = (sem_is0, sem_is1)
    sem_id = (sem_id0, sem_id1)
    sem_g = (sem_g0, sem_g1)
    sem_s = (sem_s0, sem_s1)

    c = lax.axis_index("c")
    s = lax.axis_index("s")
    r0 = s * ROWS_PER_TILE

    pltpu.sync_copy(z2.at[pl.ds(r0, ROWS_PER_TILE)],
                    acc.at[pl.ds(r0, ROWS_PER_TILE)])
    if with_counts:
      pltpu.sync_copy(z1.at[pl.ds(r0, ROWS_PER_TILE)],
                      cacc.at[pl.ds(r0, ROWS_PER_TILE)])
      pltpu.sync_copy(ones_hbm, ones)

    plsc.subcore_barrier()

    def run(feat, edges):
      base0 = s * E_PER_TILE

      def s_slice(j):
        return edges.at[0, pl.ds(base0 + j * chunk, chunk)]

      def d_slice(j):
        return edges.at[1, pl.ds(base0 + j * chunk, chunk)]

      def fire_is(j, b):
        pltpu.async_copy(s_slice(j), idx_s[b], sem_is[b])

      def wait_is(j, b):
        pltpu.make_async_copy(s_slice(j), idx_s[b], sem_is[b]).wait()

      def fire_id(j, b):
        pltpu.async_copy(d_slice(j), idx_d[b], sem_id[b])

      def wait_id(j, b):
        pltpu.make_async_copy(d_slice(j), idx_d[b], sem_id[b]).wait()

      def fire_g(b):
        pltpu.async_copy(feat.at[idx_s[b]], rows[b], sem_g[b])

      def wait_g(b):
        pltpu.make_async_copy(feat.at[idx_s[b]], rows[b], sem_g[b]).wait()

      def fire_sc(b):
        pltpu.async_copy(rows[b], acc.at[idx_d[b]], sem_s[b], add=True)
        if with_counts:
          pltpu.async_copy(ones, cacc.at[idx_d[b]], sem_c[b], add=True)

      def wait_sc(b):
        pltpu.make_async_copy(rows[b], acc.at[idx_d[b]], sem_s[b]).wait()
        if with_counts:
          pltpu.make_async_copy(ones, cacc.at[idx_d[b]], sem_c[b]).wait()

      def half(j, b, prefetch):
        wait_is(j, b)
        fire_g(b)
        if prefetch is not None:
          prefetch()
        fire_id(j, b)
        wait_g(b)
        wait_id(j, b)
        fire_sc(b)

      fire_is(0, 0)

      def pair(i, carry):
        c0 = 2 * i

        @pl.when(i > 0)
        def _():
          wait_sc(0)
        half(c0, 0, lambda: fire_is(c0 + 1, 1))

        @pl.when(i > 0)
        def _():
          wait_sc(1)

        def prefetch_next():
          @pl.when(c0 + 2 < steps)
          def _():
            fire_is(c0 + 2, 0)
        half(c0 + 1, 1, prefetch_next)
        return carry

      lax.fori_loop(0, steps // 2, pair, 0)

      if steps % 2:
        j = steps - 1
        wait_sc(0)
        half(j, 0, None)
        wait_sc(0)
        wait_sc(1)
      else:
        wait_sc(0)
        wait_sc(1)

    @pl.when(c == 0)
    def _():
      run(featp, pe)

    @pl.when(c == 1)
    def _():
      run(featn, ne)

    plsc.subcore_barrier()

    @pl.when(c == 0)
    def _():
      pltpu.sync_copy(acc.at[pl.ds(r0, ROWS_PER_TILE)],
                      sum_p.at[pl.ds(r0, ROWS_PER_TILE)])
      if with_counts:
        pltpu.sync_copy(cacc.at[pl.ds(r0, ROWS_PER_TILE)],
                        cnt_p.at[pl.ds(r0, ROWS_PER_TILE)])

    @pl.when(c == 1)
    def _():
      pltpu.sync_copy(acc.at[pl.ds(r0, ROWS_PER_TILE)],
                      sum_n.at[pl.ds(r0, ROWS_PER_TILE)])
      if with_counts:
        pltpu.sync_copy(cacc.at[pl.ds(r0, ROWS_PER_TILE)],
                        cnt_n.at[pl.ds(r0, ROWS_PER_TILE)])

  return pl.kernel(body, out_type=out_type, mesh=mesh, scratch_types=scratch,
                   compiler_params=pltpu.CompilerParams(
                       use_tc_tiling_on_sc=False))


CHUNK_32 = 1000
CHUNK_64 = 400


@functools.lru_cache(maxsize=None)
def _get_seg_sum(d_feat, chunk, with_counts):
  return _make_seg_sum(d_feat, chunk, with_counts)


def _proj1_body(x, wproj, wc, b, xp_out, xn_out, self_out):
  proj = jnp.dot(x[...], wproj[...], preferred_element_type=jnp.float32)
  xp_out[...] = proj[:, :32]
  xn_out[...] = proj[:, 32:]
  self_out[...] = (
      jnp.dot(x[...], wc[...], preferred_element_type=jnp.float32) + b[...])


def _make_proj1(rows_blk):
  grid = (N_NODES // rows_blk,)
  return pl.pallas_call(
      _proj1_body,
      grid=grid,
      in_specs=[pl.BlockSpec((rows_blk, 128), lambda i: (i, 0)),
                pl.BlockSpec((128, 64), lambda i: (0, 0)),
                pl.BlockSpec((128, 64), lambda i: (0, 0)),
                pl.BlockSpec((1, 64), lambda i: (0, 0))],
      out_specs=[pl.BlockSpec((rows_blk, 32), lambda i: (i, 0)),
                 pl.BlockSpec((rows_blk, 32), lambda i: (i, 0)),
                 pl.BlockSpec((rows_blk, 64), lambda i: (i, 0))],
      out_shape=[jax.ShapeDtypeStruct((N_NODES, 32), jnp.float32),
                 jax.ShapeDtypeStruct((N_NODES, 32), jnp.float32),
                 jax.ShapeDtypeStruct((N_NODES, 64), jnp.float32)],
  )


def _dense1_body(sp, sn, cp, cn, self1, wc, b2, z_out, self2_out):
  a = sp[...] * (1.0 / jnp.maximum(cp[...], 1.0))
  b = sn[...] * (1.0 / jnp.maximum(cn[...], 1.0))
  z = jnp.tanh(jnp.concatenate([a, b], axis=1) + self1[...])
  z_out[...] = z
  self2_out[...] = (
      jnp.dot(z, wc[...], preferred_element_type=jnp.float32) + b2[...])


def _make_dense1(rows_blk):
  grid = (N_NODES // rows_blk,)
  half = pl.BlockSpec((rows_blk, 32), lambda i: (i, 0))
  vec = pl.BlockSpec((rows_blk, 1), lambda i: (i, 0))
  full = pl.BlockSpec((rows_blk, 64), lambda i: (i, 0))
  return pl.pallas_call(
      _dense1_body,
      grid=grid,
      in_specs=[half, half, vec, vec, full,
                pl.BlockSpec((64, 64), lambda i: (0, 0)),
                pl.BlockSpec((1, 64), lambda i: (0, 0))],
      out_specs=[full, full],
      out_shape=[jax.ShapeDtypeStruct((N_NODES, 64), jnp.float32),
                 jax.ShapeDtypeStruct((N_NODES, 64), jnp.float32)],
  )


def _dense2_body(sp, sn, cp, cn, self2, wa, wb, out):
  a = sp[...] * (1.0 / jnp.maximum(cp[...], 1.0))
  b = sn[...] * (1.0 / jnp.maximum(cn[...], 1.0))
  acc = (jnp.dot(a, wa[...], preferred_element_type=jnp.float32)
         + jnp.dot(b, wb[...], preferred_element_type=jnp.float32)
         + self2[...])
  out[...] = jnp.tanh(acc)


def _make_dense2(rows_blk):
  grid = (N_NODES // rows_blk,)
  full = pl.BlockSpec((rows_blk, 64), lambda i: (i, 0))
  vec = pl.BlockSpec((rows_blk, 1), lambda i: (i, 0))
  w = pl.BlockSpec((64, 64), lambda i: (0, 0))
  return pl.pallas_call(
      _dense2_body,
      grid=grid,
      in_specs=[full, full, vec, vec, full, w, w],
      out_specs=full,
      out_shape=jax.ShapeDtypeStruct((N_NODES, 64), jnp.float32),
  )


_proj1 = _make_proj1(10000)
_dense1 = _make_dense1(10000)
_dense2 = _make_dense2(10000)

_Z2_32 = np.zeros((N_PAD, 32), np.float32)
_Z1 = np.zeros((N_PAD,), np.float32)
_Z2_64 = np.zeros((N_PAD, 64), np.float32)
_ONES_C = np.ones((CHUNK_32,), np.float32)


def kernel(x, pos_edge_index, neg_edge_index,
           W_pos1, b_pos1, W_neg1, b_neg1,
           W_pos2, b_pos2, W_neg2, b_neg2):
  pe = jnp.asarray(pos_edge_index, jnp.int32)
  ne = jnp.asarray(neg_edge_index, jnp.int32)

  h = W_pos1.shape[1]

  wproj = jnp.concatenate([W_pos1[:128], W_neg1[:128]], axis=1)
  w1c = jnp.concatenate([W_pos1[128:], W_neg1[128:]], axis=1)
  b1 = jnp.concatenate([b_pos1, b_neg1]).reshape(1, 2 * h)
  xp, xn, self1 = _proj1(x, wproj, w1c, b1)

  sum_p, sum_n, cnt_p, cnt_n = _get_seg_sum(32, CHUNK_32, True)(
      xp, xn, _Z2_32, _Z1, _ONES_C, pe, ne)
  cp = cnt_p.reshape(N_PAD, 1)
  cn = cnt_n.reshape(N_PAD, 1)

  zh = jnp.zeros((h, h), jnp.float32)
  w2c = jnp.concatenate(
      [jnp.concatenate([W_pos2[2 * h:], zh], axis=1),
       jnp.concatenate([zh, W_neg2[2 * h:]], axis=1)], axis=0)
  b2 = jnp.concatenate([b_pos2, b_neg2]).reshape(1, 2 * h)
  z, self2 = _dense1(sum_p, sum_n, cp, cn, self1, w2c, b2)

  sum_zp, sum_zn = _get_seg_sum(64, CHUNK_64, False)(
      z, z, _Z2_64, pe, ne)

  w2a = jnp.concatenate(
      [jnp.concatenate([W_pos2[:h], zh], axis=1),
       jnp.concatenate([zh, W_neg2[:h]], axis=1)], axis=0)
  w2b = jnp.concatenate(
      [jnp.concatenate([zh, W_neg2[h:2 * h]], axis=1),
       jnp.concatenate([W_pos2[h:2 * h], zh], axis=1)], axis=0)

  out = _dense2(sum_zp, sum_zn, cp, cn, self2, w2a, w2b)
  return out


# speedup vs baseline: 1.0188x; 1.0188x over previous
import functools

import jax
import jax.numpy as jnp
import numpy as np
from jax import lax
from jax.experimental import pallas as pl
from jax.experimental.pallas import tpu as pltpu
from jax.experimental.pallas import tpu_sc as plsc

N_NODES = 10000
N_PAD = 10240
N_TILES = 16
ROWS_PER_TILE = N_PAD // N_TILES
N_EDGES = 160000
E_PER_TILE = N_EDGES // N_TILES


def _make_seg_sum(d_feat, chunk, with_counts):
  steps = E_PER_TILE // chunk
  assert steps * chunk == E_PER_TILE and chunk % 8 == 0
  mesh = plsc.VectorSubcoreMesh(core_axis_name="c", subcore_axis_name="s",
                                num_cores=2, num_subcores=N_TILES)
  out_type = [jax.ShapeDtypeStruct((N_PAD, d_feat), jnp.float32),
              jax.ShapeDtypeStruct((N_PAD, d_feat), jnp.float32)]
  if with_counts:
    out_type += [jax.ShapeDtypeStruct((N_PAD,), jnp.float32),
                 jax.ShapeDtypeStruct((N_PAD,), jnp.float32)]

  scratch = [
      pltpu.VMEM((chunk,), jnp.int32),
      pltpu.VMEM((chunk,), jnp.int32),
      pltpu.VMEM((chunk,), jnp.int32),
      pltpu.VMEM((chunk,), jnp.int32),
      pltpu.VMEM((chunk, d_feat), jnp.float32),
      pltpu.VMEM((chunk, d_feat), jnp.float32),
  ]
  if with_counts:
    scratch.append(pltpu.VMEM((chunk,), jnp.float32))
  scratch.append(pltpu.VMEM_SHARED((N_PAD, d_feat), jnp.float32))
  if with_counts:
    scratch.append(pltpu.VMEM_SHARED((N_PAD,), jnp.float32))
  n_sem = 10 if with_counts else 8
  scratch += [pltpu.SemaphoreType.DMA] * n_sem

  def body(*args):
    if with_counts:
      (featp, featn, z2, z1, ones_hbm, pe, ne,
       sum_p, sum_n, cnt_p, cnt_n,
       is0, is1, id0, id1, rw0, rw1, ones, acc, cacc,
       sem_is0, sem_is1, sem_id0, sem_id1,
       sem_g0, sem_g1, sem_s0, sem_s1, sem_c0, sem_c1) = args
      sem_c = (sem_c0, sem_c1)
    else:
      (featp, featn, z2, pe, ne,
       sum_p, sum_n,
       is0, is1, id0, id1, rw0, rw1, acc,
       sem_is0, sem_is1, sem_id0, sem_id1,
       sem_g0, sem_g1, sem_s0, sem_s1) = args
    idx_s = (is0, is1)
    idx_d = (id0, id1)
    rows = (rw0, rw1)
    sem_is = (sem_is0, sem_is1)
    sem_id = (sem_id0, sem_id1)
    sem_g = (sem_g0, sem_g1)
    sem_s = (sem_s0, sem_s1)

    c = lax.axis_index("c")
    s = lax.axis_index("s")
    r0 = s * ROWS_PER_TILE

    pltpu.sync_copy(z2.at[pl.ds(r0, ROWS_PER_TILE)],
                    acc.at[pl.ds(r0, ROWS_PER_TILE)])
    if with_counts:
      pltpu.sync_copy(z1.at[pl.ds(r0, ROWS_PER_TILE)],
                      cacc.at[pl.ds(r0, ROWS_PER_TILE)])
      pltpu.sync_copy(ones_hbm, ones)

    plsc.subcore_barrier()

    def run(feat, edges):
      base0 = s * E_PER_TILE

      def s_slice(j):
        return edges.at[0, pl.ds(base0 + j * chunk, chunk)]

      def d_slice(j):
        return edges.at[1, pl.ds(base0 + j * chunk, chunk)]

      def fire_is(j, b):
        pltpu.async_copy(s_slice(j), idx_s[b], sem_is[b])

      def wait_is(j, b):
        pltpu.make_async_copy(s_slice(j), idx_s[b], sem_is[b]).wait()

      def fire_id(j, b):
        pltpu.async_copy(d_slice(j), idx_d[b], sem_id[b])

      def wait_id(j, b):
        pltpu.make_async_copy(d_slice(j), idx_d[b], sem_id[b]).wait()

      def fire_g(b):
        pltpu.async_copy(feat.at[idx_s[b]], rows[b], sem_g[b])

      def wait_g(b):
        pltpu.make_async_copy(feat.at[idx_s[b]], rows[b], sem_g[b]).wait()

      def fire_sc(b):
        pltpu.async_copy(rows[b], acc.at[idx_d[b]], sem_s[b], add=True)
        if with_counts:
          pltpu.async_copy(ones, cacc.at[idx_d[b]], sem_c[b], add=True)

      def wait_sc(b):
        pltpu.make_async_copy(rows[b], acc.at[idx_d[b]], sem_s[b]).wait()
        if with_counts:
          pltpu.make_async_copy(ones, cacc.at[idx_d[b]], sem_c[b]).wait()

      def half(j, b, prefetch):
        wait_is(j, b)
        fire_g(b)
        if prefetch is not None:
          prefetch()
        fire_id(j, b)
        wait_g(b)
        wait_id(j, b)
        fire_sc(b)

      fire_is(0, 0)

      def pair(i, carry):
        c0 = 2 * i

        @pl.when(i > 0)
        def _():
          wait_sc(0)
        half(c0, 0, lambda: fire_is(c0 + 1, 1))

        @pl.when(i > 0)
        def _():
          wait_sc(1)

        def prefetch_next():
          @pl.when(c0 + 2 < steps)
          def _():
            fire_is(c0 + 2, 0)
        half(c0 + 1, 1, prefetch_next)
        return carry

      lax.fori_loop(0, steps // 2, pair, 0)

      if steps % 2:
        j = steps - 1
        wait_sc(0)
        half(j, 0, None)
        wait_sc(0)
        wait_sc(1)
      else:
        wait_sc(0)
        wait_sc(1)

    @pl.when(c == 0)
    def _():
      run(featp, pe)

    @pl.when(c == 1)
    def _():
      run(featn, ne)

    plsc.subcore_barrier()

    @pl.when(c == 0)
    def _():
      pltpu.sync_copy(acc.at[pl.ds(r0, ROWS_PER_TILE)],
                      sum_p.at[pl.ds(r0, ROWS_PER_TILE)])
      if with_counts:
        pltpu.sync_copy(cacc.at[pl.ds(r0, ROWS_PER_TILE)],
                        cnt_p.at[pl.ds(r0, ROWS_PER_TILE)])

    @pl.when(c == 1)
    def _():
      pltpu.sync_copy(acc.at[pl.ds(r0, ROWS_PER_TILE)],
                      sum_n.at[pl.ds(r0, ROWS_PER_TILE)])
      if with_counts:
        pltpu.sync_copy(cacc.at[pl.ds(r0, ROWS_PER_TILE)],
                        cnt_n.at[pl.ds(r0, ROWS_PER_TILE)])

  return pl.kernel(body, out_type=out_type, mesh=mesh, scratch_types=scratch,
                   compiler_params=pltpu.CompilerParams(
                       use_tc_tiling_on_sc=False))


CHUNK_32 = 1000
CHUNK_64 = 400


@functools.lru_cache(maxsize=None)
def _get_seg_sum(d_feat, chunk, with_counts):
  return _make_seg_sum(d_feat, chunk, with_counts)


def _proj1_body(x, wproj, wc, b, xp_out, xn_out, self_out):
  proj = jnp.dot(x[...], wproj[...], preferred_element_type=jnp.float32)
  xp_out[...] = proj[:, :32]
  xn_out[...] = proj[:, 32:]
  self_out[...] = (
      jnp.dot(x[...], wc[...], preferred_element_type=jnp.float32) + b[...])


def _make_proj1(rows_blk):
  grid = (N_NODES // rows_blk,)
  return pl.pallas_call(
      _proj1_body,
      grid=grid,
      in_specs=[pl.BlockSpec((rows_blk, 128), lambda i: (i, 0)),
                pl.BlockSpec((128, 64), lambda i: (0, 0)),
                pl.BlockSpec((128, 64), lambda i: (0, 0)),
                pl.BlockSpec((1, 64), lambda i: (0, 0))],
      out_specs=[pl.BlockSpec((rows_blk, 32), lambda i: (i, 0)),
                 pl.BlockSpec((rows_blk, 32), lambda i: (i, 0)),
                 pl.BlockSpec((rows_blk, 64), lambda i: (i, 0))],
      out_shape=[jax.ShapeDtypeStruct((N_NODES, 32), jnp.float32),
                 jax.ShapeDtypeStruct((N_NODES, 32), jnp.float32),
                 jax.ShapeDtypeStruct((N_NODES, 64), jnp.float32)],
  )


def _dense1_body(sp, sn, cp, cn, self1, wc, b2, z_out, self2_out):
  a = sp[...] * (1.0 / jnp.maximum(cp[...], 1.0))
  b = sn[...] * (1.0 / jnp.maximum(cn[...], 1.0))
  z = jnp.tanh(jnp.concatenate([a, b], axis=1) + self1[...])
  z_out[...] = z
  self2_out[...] = (
      jnp.dot(z, wc[...], preferred_element_type=jnp.float32) + b2[...])


def _make_dense1(rows_blk):
  grid = (N_NODES // rows_blk,)
  half = pl.BlockSpec((rows_blk, 32), lambda i: (i, 0))
  vec = pl.BlockSpec((rows_blk, 1), lambda i: (i, 0))
  full = pl.BlockSpec((rows_blk, 64), lambda i: (i, 0))
  return pl.pallas_call(
      _dense1_body,
      grid=grid,
      in_specs=[half, half, vec, vec, full,
                pl.BlockSpec((64, 64), lambda i: (0, 0)),
                pl.BlockSpec((1, 64), lambda i: (0, 0))],
      out_specs=[full, full],
      out_shape=[jax.ShapeDtypeStruct((N_NODES, 64), jnp.float32),
                 jax.ShapeDtypeStruct((N_NODES, 64), jnp.float32)],
  )


def _dense2_body(sp, sn, cp, cn, self2, wa, wb, out):
  a = sp[...] * (1.0 / jnp.maximum(cp[...], 1.0))
  b = sn[...] * (1.0 / jnp.maximum(cn[...], 1.0))
  acc = (jnp.dot(a, wa[...], preferred_element_type=jnp.float32)
         + jnp.dot(b, wb[...], preferred_element_type=jnp.float32)
         + self2[...])
  out[...] = jnp.tanh(acc)


def _make_dense2(rows_blk):
  grid = (N_NODES // rows_blk,)
  full = pl.BlockSpec((rows_blk, 64), lambda i: (i, 0))
  vec = pl.BlockSpec((rows_blk, 1), lambda i: (i, 0))
  w = pl.BlockSpec((64, 64), lambda i: (0, 0))
  return pl.pallas_call(
      _dense2_body,
      grid=grid,
      in_specs=[full, full, vec, vec, full, w, w],
      out_specs=full,
      out_shape=jax.ShapeDtypeStruct((N_NODES, 64), jnp.float32),
  )


_proj1 = _make_proj1(2000)
_dense1 = _make_dense1(2000)
_dense2 = _make_dense2(2000)

_Z2_32 = np.zeros((N_PAD, 32), np.float32)
_Z1 = np.zeros((N_PAD,), np.float32)
_Z2_64 = np.zeros((N_PAD, 64), np.float32)
_ONES_C = np.ones((CHUNK_32,), np.float32)


def kernel(x, pos_edge_index, neg_edge_index,
           W_pos1, b_pos1, W_neg1, b_neg1,
           W_pos2, b_pos2, W_neg2, b_neg2):
  pe = jnp.asarray(pos_edge_index, jnp.int32)
  ne = jnp.asarray(neg_edge_index, jnp.int32)

  h = W_pos1.shape[1]

  wproj = jnp.concatenate([W_pos1[:128], W_neg1[:128]], axis=1)
  w1c = jnp.concatenate([W_pos1[128:], W_neg1[128:]], axis=1)
  b1 = jnp.concatenate([b_pos1, b_neg1]).reshape(1, 2 * h)
  xp, xn, self1 = _proj1(x, wproj, w1c, b1)

  sum_p, sum_n, cnt_p, cnt_n = _get_seg_sum(32, CHUNK_32, True)(
      xp, xn, _Z2_32, _Z1, _ONES_C, pe, ne)
  cp = cnt_p.reshape(N_PAD, 1)
  cn = cnt_n.reshape(N_PAD, 1)

  zh = jnp.zeros((h, h), jnp.float32)
  w2c = jnp.concatenate(
      [jnp.concatenate([W_pos2[2 * h:], zh], axis=1),
       jnp.concatenate([zh, W_neg2[2 * h:]], axis=1)], axis=0)
  b2 = jnp.concatenate([b_pos2, b_neg2]).reshape(1, 2 * h)
  z, self2 = _dense1(sum_p, sum_n, cp, cn, self1, w2c, b2)

  sum_zp, sum_zn = _get_seg_sum(64, CHUNK_64, False)(
      z, z, _Z2_64, pe, ne)

  w2a = jnp.concatenate(
      [jnp.concatenate([W_pos2[:h], zh], axis=1),
       jnp.concatenate([zh, W_neg2[:h]], axis=1)], axis=0)
  w2b = jnp.concatenate(
      [jnp.concatenate([zh, W_neg2[h:2 * h]], axis=1),
       jnp.concatenate([W_pos2[h:2 * h], zh], axis=1)], axis=0)

  out = _dense2(sum_zp, sum_zn, cp, cn, self2, w2a, w2b)
  return out
